# baseline (speedup 1.0000x reference)
import functools

import jax
import jax.numpy as jnp
from jax import lax
from jax.experimental import pallas as pl
from jax.experimental.pallas import tpu as pltpu
from jax.experimental.pallas import tpu_sc as plsc

B, S, D = 4, 128, 768
M = 10000
K = 5
NH = 12
DH = D // NH
V = 50257
BS = B * S

MT = 2000
NT = M // MT
NSLOT = 8
VT = 2048
NV = (V + VT - 1) // VT

_BF = jnp.bfloat16
_I32MAX = 2**31 - 1

ROWS = B * S * K


def _knn_kernel(qt_ref, mem_ref, out_ref, best_val, best_idx):
    b = pl.program_id(0)
    nt = pl.program_id(1)

    @pl.when(nt == 0)
    def _():
        best_val[...] = jnp.full((NSLOT, S), jnp.inf, jnp.float32)
        best_idx[...] = jnp.full((NSLOT, S), jnp.int32(_I32MAX), jnp.int32)

    qt = qt_ref[0]
    memf = mem_ref[0]

    dots = jnp.dot(memf.astype(_BF), qt.astype(_BF),
                   preferred_element_type=jnp.float32)
    q2 = jnp.sum(qt * qt, axis=0, keepdims=True)
    m2 = jnp.sum(memf * memf, axis=1, keepdims=True)
    dists = (q2 - 2.0 * dots) + m2

    row = lax.broadcasted_iota(jnp.int32, (MT, S), 0)
    cv = jnp.concatenate([best_val[...], dists], axis=0)
    ci = jnp.concatenate([best_idx[...], row + (b * M + nt * MT)], axis=0)

    for k in range(K):
        cur = jnp.min(cv, axis=0, keepdims=True)
        hit = cv == cur
        chosen = jnp.min(jnp.where(hit, ci, jnp.int32(_I32MAX)),
                         axis=0, keepdims=True)
        best_val[k:k + 1, :] = cur
        best_idx[k:k + 1, :] = chosen
        cv = jnp.where(hit & (ci == chosen), jnp.inf, cv)

    @pl.when(nt == NT - 1)
    def _():
        out_ref[0] = best_idx[...]


def _knn_topk(qt, memory):
    return pl.pallas_call(
        _knn_kernel,
        grid=(B, NT),
        in_specs=[
            pl.BlockSpec((1, D, S), lambda b, nt: (b, 0, 0)),
            pl.BlockSpec((1, MT, D), lambda b, nt: (b, nt, 0)),
        ],
        out_specs=pl.BlockSpec((1, NSLOT, S), lambda b, nt: (b, 0, 0)),
        out_shape=jax.ShapeDtypeStruct((B, NSLOT, S), jnp.int32),
        scratch_shapes=[
            pltpu.VMEM((NSLOT, S), jnp.float32),
            pltpu.VMEM((NSLOT, S), jnp.int32),
        ],
        compiler_params=pltpu.CompilerParams(
            dimension_semantics=("arbitrary", "arbitrary")),
    )(qt, memory)


def _gather_rows(idx_flat, mem_flat):
    info = plsc.get_sparse_core_info()
    _NC = info.num_cores
    _NW = info.num_cores * info.num_subcores
    RPW = ROWS // _NW
    mesh = plsc.VectorSubcoreMesh(core_axis_name="c", subcore_axis_name="s")

    @functools.partial(
        pl.kernel,
        out_type=jax.ShapeDtypeStruct((ROWS, D), jnp.float32),
        mesh=mesh,
        compiler_params=pltpu.CompilerParams(use_tc_tiling_on_sc=True),
        scratch_types=[
            pltpu.VMEM((RPW,), jnp.int32),
            pltpu.VMEM((RPW, D), jnp.float32),
            pltpu.SemaphoreType.DMA,
        ],
    )
    def k(idx_hbm, table_hbm, out_hbm, idx_v, rows_v, sem):
        wid = lax.axis_index("s") * _NC + lax.axis_index("c")
        base = wid * RPW
        pltpu.sync_copy(idx_hbm.at[pl.ds(base, RPW)], idx_v)
        pltpu.async_copy(table_hbm.at[idx_v], rows_v, sem).wait()
        pltpu.sync_copy(rows_v, out_hbm.at[pl.ds(base, RPW)])

    return k(idx_flat, mem_flat)


def _ln(x, g, b):
    mu = jnp.mean(x, axis=-1, keepdims=True)
    v = jnp.mean((x - mu) * (x - mu), axis=-1, keepdims=True)
    return (x - mu) / jnp.sqrt(v + 1e-5) * g + b


def _fc_kernel(q_ref, r_ref, w_ref, b_ref, o_ref):
    wq = w_ref[0:D, :].astype(_BF)
    wr = w_ref[D:, :].astype(_BF)
    acc = jnp.dot(q_ref[...].astype(_BF), wq, preferred_element_type=jnp.float32)
    acc = acc + jnp.dot(r_ref[...].astype(_BF), wr,
                        preferred_element_type=jnp.float32)
    o_ref[...] = acc + b_ref[...]


def _fc(q2d, r2d, w, bias):
    return pl.pallas_call(
        _fc_kernel,
        out_shape=jax.ShapeDtypeStruct((BS, D), jnp.float32),
    )(q2d, r2d, w, bias)


def _attn_kernel(x_ref, g1, be1, wqkv, bqkv, wo, bo, o_ref, osc):
    x = x_ref[0]
    h = _ln(x, g1[...], be1[...])
    qkv = jnp.dot(h.astype(_BF), wqkv[...].astype(_BF),
                  preferred_element_type=jnp.float32) + bqkv[...]
    ri = lax.broadcasted_iota(jnp.int32, (S, S), 0)
    cij = lax.broadcasted_iota(jnp.int32, (S, S), 1)
    causal = ri >= cij
    for i in range(NH):
        q = qkv[:, i * DH:(i + 1) * DH]
        kk = qkv[:, D + i * DH:D + (i + 1) * DH]
        v = qkv[:, 2 * D + i * DH:2 * D + (i + 1) * DH]
        att = lax.dot_general(q.astype(_BF), kk.astype(_BF),
                              (((1,), (1,)), ((), ())),
                              preferred_element_type=jnp.float32) / 8.0
        att = jnp.where(causal, att, jnp.float32(-1e9))
        att = att - jnp.max(att, axis=-1, keepdims=True)
        e = jnp.exp(att)
        p = e / jnp.sum(e, axis=-1, keepdims=True)
        osc[:, i * DH:(i + 1) * DH] = jnp.dot(
            p.astype(_BF), v.astype(_BF), preferred_element_type=jnp.float32)
    o_ref[0] = x + jnp.dot(osc[...].astype(_BF), wo[...].astype(_BF),
                           preferred_element_type=jnp.float32) + bo[...]


def _mlp_kernel(x_ref, g2, be2, w1, b1, w2, b2, o_ref):
    x = x_ref[0]
    h = _ln(x, g2[...], be2[...])
    hh = jnp.dot(h.astype(_BF), w1[...].astype(_BF),
                 preferred_element_type=jnp.float32) + b1[...]
    hh = jax.nn.gelu(hh)
    o_ref[0] = x + jnp.dot(hh.astype(_BF), w2[...].astype(_BF),
                           preferred_element_type=jnp.float32) + b2[...]


def _block(x, p):
    r1 = jnp.reshape
    x = pl.pallas_call(
        _attn_kernel,
        grid=(B,),
        in_specs=[
            pl.BlockSpec((1, S, D), lambda b: (b, 0, 0)),
            pl.BlockSpec((1, D), lambda b: (0, 0)),
            pl.BlockSpec((1, D), lambda b: (0, 0)),
            pl.BlockSpec((D, 3 * D), lambda b: (0, 0)),
            pl.BlockSpec((1, 3 * D), lambda b: (0, 0)),
            pl.BlockSpec((D, D), lambda b: (0, 0)),
            pl.BlockSpec((1, D), lambda b: (0, 0)),
        ],
        out_specs=pl.BlockSpec((1, S, D), lambda b: (b, 0, 0)),
        out_shape=jax.ShapeDtypeStruct((B, S, D), jnp.float32),
        scratch_shapes=[pltpu.VMEM((S, D), jnp.float32)],
        compiler_params=pltpu.CompilerParams(
            dimension_semantics=("arbitrary",)),
    )(x, r1(p["ln1_g"], (1, D)), r1(p["ln1_b"], (1, D)),
      p["Wqkv"], r1(p["bqkv"], (1, 3 * D)), p["Wo"], r1(p["bo"], (1, D)))

    x = pl.pallas_call(
        _mlp_kernel,
        grid=(B,),
        in_specs=[
            pl.BlockSpec((1, S, D), lambda b: (b, 0, 0)),
            pl.BlockSpec((1, D), lambda b: (0, 0)),
            pl.BlockSpec((1, D), lambda b: (0, 0)),
            pl.BlockSpec((D, 4 * D), lambda b: (0, 0)),
            pl.BlockSpec((1, 4 * D), lambda b: (0, 0)),
            pl.BlockSpec((4 * D, D), lambda b: (0, 0)),
            pl.BlockSpec((1, D), lambda b: (0, 0)),
        ],
        out_specs=pl.BlockSpec((1, S, D), lambda b: (b, 0, 0)),
        out_shape=jax.ShapeDtypeStruct((B, S, D), jnp.float32),
        compiler_params=pltpu.CompilerParams(
            dimension_semantics=("arbitrary",)),
    )(x, r1(p["ln2_g"], (1, D)), r1(p["ln2_b"], (1, D)),
      p["W1"], r1(p["b1"], (1, 4 * D)), p["W2"], r1(p["b2"], (1, D)))
    return x


def _lm_kernel(x_ref, g, bb, w_ref, o_ref):
    h = _ln(x_ref[...], g[...], bb[...])
    o_ref[...] = jnp.dot(h.astype(_BF), w_ref[...].astype(_BF),
                         preferred_element_type=jnp.float32)


def _lm_head(x2d, g, bb, wlm):
    return pl.pallas_call(
        _lm_kernel,
        grid=(NV,),
        in_specs=[
            pl.BlockSpec((BS, D), lambda j: (0, 0)),
            pl.BlockSpec((1, D), lambda j: (0, 0)),
            pl.BlockSpec((1, D), lambda j: (0, 0)),
            pl.BlockSpec((D, VT), lambda j: (0, j)),
        ],
        out_specs=pl.BlockSpec((BS, VT), lambda j: (0, j)),
        out_shape=jax.ShapeDtypeStruct((BS, V), jnp.float32),
        compiler_params=pltpu.CompilerParams(
            dimension_semantics=("arbitrary",)),
    )(x2d, g, bb, wlm)


def kernel(input_vectors, memory, params):
    p = params
    qt = jnp.transpose(input_vectors, (0, 2, 1))
    idx_pad = _knn_topk(qt, memory)
    idx_flat = idx_pad[:, :K, :].transpose(0, 2, 1).reshape(ROWS)

    retrieved = _gather_rows(idx_flat, memory.reshape(B * M, D))
    r2d = retrieved.reshape(BS, K * D)
    q2d = input_vectors.reshape(BS, D)

    x = _fc(q2d, r2d, p["W_fc"], p["b_fc"].reshape(1, D)).reshape(B, S, D)
    for bp in p["blocks"]:
        x = _block(x, bp)

    logits = _lm_head(x.reshape(BS, D), p["lnf_g"].reshape(1, D),
                      p["lnf_b"].reshape(1, D), p["W_lm"])
    return logits.reshape(B, S, V)


# speedup vs baseline: 1.7201x; 1.7201x over previous
import functools

import jax
import jax.numpy as jnp
from jax import lax
from jax.experimental import pallas as pl
from jax.experimental.pallas import tpu as pltpu
from jax.experimental.pallas import tpu_sc as plsc

B, S, D = 4, 128, 768
M = 10000
K = 5
NH = 12
DH = D // NH
V = 50257
BS = B * S

MT = 2000
NT = M // MT
NSLOT = 8
VT = 2048
NV = (V + VT - 1) // VT

_BF = jnp.bfloat16
_I32MAX = 2**31 - 1

ROWS = B * S * K


def _knn_kernel(qt_ref, mem_ref, out_ref, best_val, best_idx):
    b = pl.program_id(0)
    nt = pl.program_id(1)

    @pl.when(nt == 0)
    def _():
        best_val[...] = jnp.full((NSLOT, S), jnp.inf, jnp.float32)
        best_idx[...] = jnp.full((NSLOT, S), jnp.int32(_I32MAX), jnp.int32)

    qt = qt_ref[0]
    memf = mem_ref[0]

    dots = jnp.dot(memf.astype(_BF), qt.astype(_BF),
                   preferred_element_type=jnp.float32)
    q2 = jnp.sum(qt * qt, axis=0, keepdims=True)
    m2 = jnp.sum(memf * memf, axis=1, keepdims=True)
    dists = (q2 - 2.0 * dots) + m2

    row = lax.broadcasted_iota(jnp.int32, (MT, S), 0)
    cv = jnp.concatenate([best_val[...], dists], axis=0)
    ci = jnp.concatenate([best_idx[...], row + (b * M + nt * MT)], axis=0)

    for k in range(K):
        cur = jnp.min(cv, axis=0, keepdims=True)
        hit = cv == cur
        chosen = jnp.min(jnp.where(hit, ci, jnp.int32(_I32MAX)),
                         axis=0, keepdims=True)
        best_val[k:k + 1, :] = cur
        best_idx[k:k + 1, :] = chosen
        cv = jnp.where(hit & (ci == chosen), jnp.inf, cv)

    @pl.when(nt == NT - 1)
    def _():
        out_ref[0] = best_idx[...]


def _knn_topk(qt, memory):
    return pl.pallas_call(
        _knn_kernel,
        grid=(B, NT),
        in_specs=[
            pl.BlockSpec((1, D, S), lambda b, nt: (b, 0, 0)),
            pl.BlockSpec((1, MT, D), lambda b, nt: (b, nt, 0)),
        ],
        out_specs=pl.BlockSpec((1, NSLOT, S), lambda b, nt: (b, 0, 0)),
        out_shape=jax.ShapeDtypeStruct((B, NSLOT, S), jnp.int32),
        scratch_shapes=[
            pltpu.VMEM((NSLOT, S), jnp.float32),
            pltpu.VMEM((NSLOT, S), jnp.int32),
        ],
        compiler_params=pltpu.CompilerParams(
            dimension_semantics=("arbitrary", "arbitrary")),
    )(qt, memory)


def _gather_rows(idx_flat, mem_flat):
    info = plsc.get_sparse_core_info()
    _NC = info.num_cores
    _NW = info.num_cores * info.num_subcores
    RPW = ROWS // _NW
    mesh = plsc.VectorSubcoreMesh(core_axis_name="c", subcore_axis_name="s")

    @functools.partial(
        pl.kernel,
        out_type=jax.ShapeDtypeStruct((ROWS, D), jnp.float32),
        mesh=mesh,
        compiler_params=pltpu.CompilerParams(use_tc_tiling_on_sc=True),
        scratch_types=[
            pltpu.VMEM((RPW,), jnp.int32),
            pltpu.VMEM((RPW, D), jnp.float32),
            pltpu.SemaphoreType.DMA,
        ],
    )
    def k(idx_hbm, table_hbm, out_hbm, idx_v, rows_v, sem):
        wid = lax.axis_index("s") * _NC + lax.axis_index("c")
        base = wid * RPW
        pltpu.sync_copy(idx_hbm.at[pl.ds(base, RPW)], idx_v)
        pltpu.async_copy(table_hbm.at[idx_v], rows_v, sem).wait()
        pltpu.sync_copy(rows_v, out_hbm.at[pl.ds(base, RPW)])

    return k(idx_flat, mem_flat)


def _ln(x, g, b):
    mu = jnp.mean(x, axis=-1, keepdims=True)
    v = jnp.mean((x - mu) * (x - mu), axis=-1, keepdims=True)
    return (x - mu) / jnp.sqrt(v + 1e-5) * g + b


def _fc_kernel(q_ref, r_ref, w_ref, b_ref, o_ref):
    wq = w_ref[0:D, :].astype(_BF)
    wr = w_ref[D:, :].astype(_BF)
    acc = jnp.dot(q_ref[...].astype(_BF), wq, preferred_element_type=jnp.float32)
    acc = acc + jnp.dot(r_ref[...].astype(_BF), wr,
                        preferred_element_type=jnp.float32)
    o_ref[...] = acc + b_ref[...]


def _fc(q2d, r2d, w, bias):
    return pl.pallas_call(
        _fc_kernel,
        out_shape=jax.ShapeDtypeStruct((BS, D), jnp.float32),
    )(q2d, r2d, w, bias)


def _attn_kernel(x_ref, g1, be1, wqkv, bqkv, wo, bo, o_ref, osc):
    x = x_ref[0]
    h = _ln(x, g1[...], be1[...])
    qkv = jnp.dot(h.astype(_BF), wqkv[...].astype(_BF),
                  preferred_element_type=jnp.float32) + bqkv[...]
    ri = lax.broadcasted_iota(jnp.int32, (S, S), 0)
    cij = lax.broadcasted_iota(jnp.int32, (S, S), 1)
    causal = ri >= cij
    for i in range(NH):
        q = qkv[:, i * DH:(i + 1) * DH]
        kk = qkv[:, D + i * DH:D + (i + 1) * DH]
        v = qkv[:, 2 * D + i * DH:2 * D + (i + 1) * DH]
        att = lax.dot_general(q.astype(_BF), kk.astype(_BF),
                              (((1,), (1,)), ((), ())),
                              preferred_element_type=jnp.float32) / 8.0
        att = jnp.where(causal, att, jnp.float32(-1e9))
        att = att - jnp.max(att, axis=-1, keepdims=True)
        e = jnp.exp(att)
        p = e / jnp.sum(e, axis=-1, keepdims=True)
        osc[:, i * DH:(i + 1) * DH] = jnp.dot(
            p.astype(_BF), v.astype(_BF), preferred_element_type=jnp.float32)
    o_ref[0] = x + jnp.dot(osc[...].astype(_BF), wo[...].astype(_BF),
                           preferred_element_type=jnp.float32) + bo[...]


def _mlp_kernel(x_ref, g2, be2, w1, b1, w2, b2, o_ref):
    x = x_ref[0]
    h = _ln(x, g2[...], be2[...])
    hh = jnp.dot(h.astype(_BF), w1[...].astype(_BF),
                 preferred_element_type=jnp.float32) + b1[...]
    hh = jax.nn.gelu(hh)
    o_ref[0] = x + jnp.dot(hh.astype(_BF), w2[...].astype(_BF),
                           preferred_element_type=jnp.float32) + b2[...]


def _block(x, p):
    r1 = jnp.reshape
    x = pl.pallas_call(
        _attn_kernel,
        grid=(B,),
        in_specs=[
            pl.BlockSpec((1, S, D), lambda b: (b, 0, 0)),
            pl.BlockSpec((1, D), lambda b: (0, 0)),
            pl.BlockSpec((1, D), lambda b: (0, 0)),
            pl.BlockSpec((D, 3 * D), lambda b: (0, 0)),
            pl.BlockSpec((1, 3 * D), lambda b: (0, 0)),
            pl.BlockSpec((D, D), lambda b: (0, 0)),
            pl.BlockSpec((1, D), lambda b: (0, 0)),
        ],
        out_specs=pl.BlockSpec((1, S, D), lambda b: (b, 0, 0)),
        out_shape=jax.ShapeDtypeStruct((B, S, D), jnp.float32),
        scratch_shapes=[pltpu.VMEM((S, D), jnp.float32)],
        compiler_params=pltpu.CompilerParams(
            dimension_semantics=("arbitrary",)),
    )(x, r1(p["ln1_g"], (1, D)), r1(p["ln1_b"], (1, D)),
      p["Wqkv"], r1(p["bqkv"], (1, 3 * D)), p["Wo"], r1(p["bo"], (1, D)))

    x = pl.pallas_call(
        _mlp_kernel,
        grid=(B,),
        in_specs=[
            pl.BlockSpec((1, S, D), lambda b: (b, 0, 0)),
            pl.BlockSpec((1, D), lambda b: (0, 0)),
            pl.BlockSpec((1, D), lambda b: (0, 0)),
            pl.BlockSpec((D, 4 * D), lambda b: (0, 0)),
            pl.BlockSpec((1, 4 * D), lambda b: (0, 0)),
            pl.BlockSpec((4 * D, D), lambda b: (0, 0)),
            pl.BlockSpec((1, D), lambda b: (0, 0)),
        ],
        out_specs=pl.BlockSpec((1, S, D), lambda b: (b, 0, 0)),
        out_shape=jax.ShapeDtypeStruct((B, S, D), jnp.float32),
        compiler_params=pltpu.CompilerParams(
            dimension_semantics=("arbitrary",)),
    )(x, r1(p["ln2_g"], (1, D)), r1(p["ln2_b"], (1, D)),
      p["W1"], r1(p["b1"], (1, 4 * D)), p["W2"], r1(p["b2"], (1, D)))
    return x


def _lm_kernel(x_ref, g, bb, wt_ref, o_ref):
    h = _ln(x_ref[...], g[...], bb[...])
    ot = lax.dot_general(wt_ref[...].astype(_BF), h.astype(_BF),
                         (((1,), (1,)), ((), ())),
                         preferred_element_type=jnp.float32)
    o_ref[...] = ot.reshape(VT, B, S)


def _lm_head(x2d, g, bb, wlm_t):
    return pl.pallas_call(
        _lm_kernel,
        grid=(NV,),
        in_specs=[
            pl.BlockSpec((BS, D), lambda j: (0, 0)),
            pl.BlockSpec((1, D), lambda j: (0, 0)),
            pl.BlockSpec((1, D), lambda j: (0, 0)),
            pl.BlockSpec((VT, D), lambda j: (j, 0)),
        ],
        out_specs=pl.BlockSpec((VT, B, S), lambda j: (j, 0, 0)),
        out_shape=jax.ShapeDtypeStruct((V, B, S), jnp.float32),
        compiler_params=pltpu.CompilerParams(
            dimension_semantics=("arbitrary",)),
    )(x2d, g, bb, wlm_t)


def kernel(input_vectors, memory, params):
    p = params
    qt = jnp.transpose(input_vectors, (0, 2, 1))
    idx_pad = _knn_topk(qt, memory)
    idx_flat = idx_pad[:, :K, :].transpose(0, 2, 1).reshape(ROWS)

    retrieved = _gather_rows(idx_flat, memory.reshape(B * M, D))
    r2d = retrieved.reshape(BS, K * D)
    q2d = input_vectors.reshape(BS, D)

    x = _fc(q2d, r2d, p["W_fc"], p["b_fc"].reshape(1, D)).reshape(B, S, D)
    for bp in p["blocks"]:
        x = _block(x, bp)

    logits_t = _lm_head(x.reshape(BS, D), p["lnf_g"].reshape(1, D),
                        p["lnf_b"].reshape(1, D),
                        jnp.transpose(p["W_lm"]))
    return jnp.transpose(logits_t, (1, 2, 0))


# speedup vs baseline: 1.9967x; 1.1608x over previous
import functools

import jax
import jax.numpy as jnp
from jax import lax
from jax.experimental import pallas as pl
from jax.experimental.pallas import tpu as pltpu
from jax.experimental.pallas import tpu_sc as plsc

B, S, D = 4, 128, 768
M = 10000
K = 5
NH = 12
DH = D // NH
V = 50257
BS = B * S

MT = 2000
NT = M // MT
NSLOT = 8
VT = 4096
NV = (V + VT - 1) // VT

_BF = jnp.bfloat16
_I32MAX = 2**31 - 1

ROWS = B * S * K


def _knn_kernel(qt_ref, mem_ref, out_ref, best_val, best_idx):
    b = pl.program_id(0)
    nt = pl.program_id(1)

    @pl.when(nt == 0)
    def _():
        best_val[...] = jnp.full((NSLOT, S), jnp.inf, jnp.float32)
        best_idx[...] = jnp.full((NSLOT, S), jnp.int32(_I32MAX), jnp.int32)

    qt = qt_ref[0]
    memf = mem_ref[0]

    dots = jnp.dot(memf.astype(_BF), qt.astype(_BF),
                   preferred_element_type=jnp.float32)
    q2 = jnp.sum(qt * qt, axis=0, keepdims=True)
    m2 = jnp.sum(memf * memf, axis=1, keepdims=True)
    dists = (q2 - 2.0 * dots) + m2

    row = lax.broadcasted_iota(jnp.int32, (MT, S), 0)
    cv = jnp.concatenate([best_val[...], dists], axis=0)
    ci = jnp.concatenate([best_idx[...], row + (b * M + nt * MT)], axis=0)

    for k in range(K):
        cur = jnp.min(cv, axis=0, keepdims=True)
        hit = cv == cur
        chosen = jnp.min(jnp.where(hit, ci, jnp.int32(_I32MAX)),
                         axis=0, keepdims=True)
        best_val[k:k + 1, :] = cur
        best_idx[k:k + 1, :] = chosen
        cv = jnp.where(hit & (ci == chosen), jnp.inf, cv)

    @pl.when(nt == NT - 1)
    def _():
        out_ref[0] = best_idx[...]


def _knn_topk(qt, memory):
    return pl.pallas_call(
        _knn_kernel,
        grid=(B, NT),
        in_specs=[
            pl.BlockSpec((1, D, S), lambda b, nt: (b, 0, 0)),
            pl.BlockSpec((1, MT, D), lambda b, nt: (b, nt, 0)),
        ],
        out_specs=pl.BlockSpec((1, NSLOT, S), lambda b, nt: (b, 0, 0)),
        out_shape=jax.ShapeDtypeStruct((B, NSLOT, S), jnp.int32),
        scratch_shapes=[
            pltpu.VMEM((NSLOT, S), jnp.float32),
            pltpu.VMEM((NSLOT, S), jnp.int32),
        ],
        compiler_params=pltpu.CompilerParams(
            dimension_semantics=("arbitrary", "arbitrary")),
    )(qt, memory)


def _gather_rows(idx_flat, mem_flat):
    info = plsc.get_sparse_core_info()
    _NC = info.num_cores
    _NW = info.num_cores * info.num_subcores
    RPW = ROWS // _NW
    mesh = plsc.VectorSubcoreMesh(core_axis_name="c", subcore_axis_name="s")

    @functools.partial(
        pl.kernel,
        out_type=jax.ShapeDtypeStruct((ROWS, D), jnp.float32),
        mesh=mesh,
        compiler_params=pltpu.CompilerParams(use_tc_tiling_on_sc=True),
        scratch_types=[
            pltpu.VMEM((RPW,), jnp.int32),
            pltpu.VMEM((RPW, D), jnp.float32),
            pltpu.SemaphoreType.DMA,
        ],
    )
    def k(idx_hbm, table_hbm, out_hbm, idx_v, rows_v, sem):
        wid = lax.axis_index("s") * _NC + lax.axis_index("c")
        base = wid * RPW
        pltpu.sync_copy(idx_hbm.at[pl.ds(base, RPW)], idx_v)
        pltpu.async_copy(table_hbm.at[idx_v], rows_v, sem).wait()
        pltpu.sync_copy(rows_v, out_hbm.at[pl.ds(base, RPW)])

    return k(idx_flat, mem_flat)


def _ln(x, g, b):
    mu = jnp.mean(x, axis=-1, keepdims=True)
    v = jnp.mean((x - mu) * (x - mu), axis=-1, keepdims=True)
    return (x - mu) / jnp.sqrt(v + 1e-5) * g + b


def _fc_kernel(q_ref, r_ref, w_ref, b_ref, o_ref):
    wq = w_ref[0:D, :].astype(_BF)
    acc = jnp.dot(q_ref[...].astype(_BF), wq, preferred_element_type=jnp.float32)
    for k in range(K):
        wr = w_ref[(1 + k) * D:(2 + k) * D, :].astype(_BF)
        acc = acc + jnp.dot(r_ref[k * BS:(k + 1) * BS, :].astype(_BF), wr,
                            preferred_element_type=jnp.float32)
    o_ref[...] = acc + b_ref[...]


def _fc(q2d, r2d, w, bias):
    return pl.pallas_call(
        _fc_kernel,
        out_shape=jax.ShapeDtypeStruct((BS, D), jnp.float32),
    )(q2d, r2d, w, bias)


def _attn_kernel(x_ref, g1, be1, wqkv, bqkv, wo, bo, o_ref, att_sc, osc):
    x = x_ref[...]
    h = _ln(x, g1[...], be1[...])
    qkv = jnp.dot(h.astype(_BF), wqkv[...].astype(_BF),
                  preferred_element_type=jnp.float32) + bqkv[...]
    qkv_b = qkv.astype(_BF)

    for i in range(NH):
        q = qkv_b[:, i * DH:(i + 1) * DH]
        kk = qkv_b[:, D + i * DH:D + (i + 1) * DH]
        full = lax.dot_general(q, kk, (((1,), (1,)), ((), ())),
                               preferred_element_type=jnp.float32)
        for b in range(B):
            att_sc[b * S:(b + 1) * S, i * S:(i + 1) * S] = (
                full[b * S:(b + 1) * S, b * S:(b + 1) * S])

    ri = lax.broadcasted_iota(jnp.int32, (BS, S), 0)
    cij = lax.broadcasted_iota(jnp.int32, (BS, S), 1)
    causal = (ri % S) >= cij
    for i in range(NH):
        att = att_sc[:, i * S:(i + 1) * S] / 8.0
        att = jnp.where(causal, att, jnp.float32(-1e9))
        att = att - jnp.max(att, axis=-1, keepdims=True)
        e = jnp.exp(att)
        att_sc[:, i * S:(i + 1) * S] = e / jnp.sum(e, axis=-1, keepdims=True)

    for i in range(NH):
        for b in range(B):
            p = att_sc[b * S:(b + 1) * S, i * S:(i + 1) * S].astype(_BF)
            v = qkv_b[b * S:(b + 1) * S, 2 * D + i * DH:2 * D + (i + 1) * DH]
            osc[b * S:(b + 1) * S, i * DH:(i + 1) * DH] = jnp.dot(
                p, v, preferred_element_type=jnp.float32)

    o_ref[...] = x + jnp.dot(osc[...].astype(_BF), wo[...].astype(_BF),
                             preferred_element_type=jnp.float32) + bo[...]


def _mlp_kernel(x_ref, g2, be2, w1, b1, w2, b2, o_ref):
    x = x_ref[...]
    h = _ln(x, g2[...], be2[...])
    hh = jnp.dot(h.astype(_BF), w1[...].astype(_BF),
                 preferred_element_type=jnp.float32) + b1[...]
    hh = jax.nn.gelu(hh)
    o_ref[...] = x + jnp.dot(hh.astype(_BF), w2[...].astype(_BF),
                             preferred_element_type=jnp.float32) + b2[...]


def _block(x, p):
    r1 = jnp.reshape
    x = pl.pallas_call(
        _attn_kernel,
        out_shape=jax.ShapeDtypeStruct((BS, D), jnp.float32),
        scratch_shapes=[pltpu.VMEM((BS, NH * S), jnp.float32),
                        pltpu.VMEM((BS, D), jnp.float32)],
    )(x, r1(p["ln1_g"], (1, D)), r1(p["ln1_b"], (1, D)),
      p["Wqkv"], r1(p["bqkv"], (1, 3 * D)), p["Wo"], r1(p["bo"], (1, D)))

    x = pl.pallas_call(
        _mlp_kernel,
        out_shape=jax.ShapeDtypeStruct((BS, D), jnp.float32),
    )(x, r1(p["ln2_g"], (1, D)), r1(p["ln2_b"], (1, D)),
      p["W1"], r1(p["b1"], (1, 4 * D)), p["W2"], r1(p["b2"], (1, D)))
    return x


def _lm_kernel(x_ref, g, bb, wt_ref, o_ref):
    h = _ln(x_ref[...], g[...], bb[...])
    ot = lax.dot_general(wt_ref[...].astype(_BF), h.astype(_BF),
                         (((1,), (1,)), ((), ())),
                         preferred_element_type=jnp.float32)
    o_ref[...] = ot.reshape(VT, B, S)


def _lm_head(x2d, g, bb, wlm_t):
    return pl.pallas_call(
        _lm_kernel,
        grid=(NV,),
        in_specs=[
            pl.BlockSpec((BS, D), lambda j: (0, 0)),
            pl.BlockSpec((1, D), lambda j: (0, 0)),
            pl.BlockSpec((1, D), lambda j: (0, 0)),
            pl.BlockSpec((VT, D), lambda j: (j, 0)),
        ],
        out_specs=pl.BlockSpec((VT, B, S), lambda j: (j, 0, 0)),
        out_shape=jax.ShapeDtypeStruct((V, B, S), jnp.float32),
        compiler_params=pltpu.CompilerParams(
            dimension_semantics=("arbitrary",)),
    )(x2d, g, bb, wlm_t)


def kernel(input_vectors, memory, params):
    p = params
    qt = jnp.transpose(input_vectors, (0, 2, 1))
    idx_pad = _knn_topk(qt, memory)
    idx_flat = idx_pad[:, :K, :].transpose(1, 0, 2).reshape(ROWS)

    retrieved = _gather_rows(idx_flat, memory.reshape(B * M, D))
    q2d = input_vectors.reshape(BS, D)

    x = _fc(q2d, retrieved, p["W_fc"], p["b_fc"].reshape(1, D))
    for bp in p["blocks"]:
        x = _block(x, bp)

    logits_t = _lm_head(x, p["lnf_g"].reshape(1, D),
                        p["lnf_b"].reshape(1, D),
                        jnp.transpose(p["W_lm"]))
    return jnp.transpose(logits_t, (1, 2, 0))


# speedup vs baseline: 2.0765x; 1.0399x over previous
import functools

import jax
import jax.numpy as jnp
from jax import lax
from jax.experimental import pallas as pl
from jax.experimental.pallas import tpu as pltpu
from jax.experimental.pallas import tpu_sc as plsc

B, S, D = 4, 128, 768
M = 10000
K = 5
NH = 12
DH = D // NH
V = 50257
BS = B * S

MT = 5000
NT = M // MT
NSLOT = 8
VT = 4096
NV = (V + VT - 1) // VT

_BF = jnp.bfloat16
_I32MAX = 2**31 - 1

ROWS = B * S * K


def _knn_kernel(q_ref, mem_ref, out_ref, best_val, best_idx, q2s):
    b = pl.program_id(0)
    nt = pl.program_id(1)

    @pl.when(nt == 0)
    def _():
        best_val[...] = jnp.full((NSLOT, S), jnp.inf, jnp.float32)
        best_idx[...] = jnp.full((NSLOT, S), jnp.int32(_I32MAX), jnp.int32)
        q0 = q_ref[0]
        q2s[...] = jnp.transpose(jnp.sum(q0 * q0, axis=1, keepdims=True))

    q = q_ref[0]
    memf = mem_ref[0]

    qn = (q * jnp.float32(-2.0)).astype(_BF)
    dots2 = lax.dot_general(memf.astype(_BF), qn,
                            (((1,), (1,)), ((), ())),
                            preferred_element_type=jnp.float32)
    m2 = jnp.sum(memf * memf, axis=1, keepdims=True)
    dists = (q2s[...] + dots2) + m2

    rowi = lax.broadcasted_iota(jnp.int32, (MT, S), 0) + (b * M + nt * MT)
    cv = dists
    bv = best_val[...]
    bi = best_idx[...]
    newv, newi = [], []
    for k in range(K):
        cur = jnp.minimum(jnp.min(cv, axis=0, keepdims=True),
                          jnp.min(bv, axis=0, keepdims=True))
        hit_d = cv == cur
        hit_b = bv == cur
        cand = jnp.minimum(
            jnp.min(jnp.where(hit_d, rowi, jnp.int32(_I32MAX)),
                    axis=0, keepdims=True),
            jnp.min(jnp.where(hit_b, bi, jnp.int32(_I32MAX)),
                    axis=0, keepdims=True))
        newv.append(cur)
        newi.append(cand)
        cv = jnp.where(hit_d & (rowi == cand), jnp.inf, cv)
        bv = jnp.where(hit_b & (bi == cand), jnp.inf, bv)

    pad = jnp.full((NSLOT - K, S), jnp.inf, jnp.float32)
    padi = jnp.full((NSLOT - K, S), jnp.int32(_I32MAX), jnp.int32)
    best_val[...] = jnp.concatenate(newv + [pad], axis=0)
    best_idx[...] = jnp.concatenate(newi + [padi], axis=0)

    @pl.when(nt == NT - 1)
    def _():
        out_ref[0] = best_idx[...]


def _knn_topk(q, memory):
    return pl.pallas_call(
        _knn_kernel,
        grid=(B, NT),
        in_specs=[
            pl.BlockSpec((1, S, D), lambda b, nt: (b, 0, 0)),
            pl.BlockSpec((1, MT, D), lambda b, nt: (b, nt, 0)),
        ],
        out_specs=pl.BlockSpec((1, NSLOT, S), lambda b, nt: (b, 0, 0)),
        out_shape=jax.ShapeDtypeStruct((B, NSLOT, S), jnp.int32),
        scratch_shapes=[
            pltpu.VMEM((NSLOT, S), jnp.float32),
            pltpu.VMEM((NSLOT, S), jnp.int32),
            pltpu.VMEM((1, S), jnp.float32),
        ],
        compiler_params=pltpu.CompilerParams(
            dimension_semantics=("arbitrary", "arbitrary")),
    )(q, memory)


def _gather_rows(idx_flat, mem_flat):
    info = plsc.get_sparse_core_info()
    _NC = info.num_cores
    _NW = info.num_cores * info.num_subcores
    RPW = ROWS // _NW
    mesh = plsc.VectorSubcoreMesh(core_axis_name="c", subcore_axis_name="s")

    @functools.partial(
        pl.kernel,
        out_type=jax.ShapeDtypeStruct((ROWS, D), jnp.float32),
        mesh=mesh,
        compiler_params=pltpu.CompilerParams(use_tc_tiling_on_sc=True),
        scratch_types=[
            pltpu.VMEM((RPW,), jnp.int32),
            pltpu.VMEM((RPW, D), jnp.float32),
            pltpu.SemaphoreType.DMA,
        ],
    )
    def k(idx_hbm, table_hbm, out_hbm, idx_v, rows_v, sem):
        wid = lax.axis_index("s") * _NC + lax.axis_index("c")
        base = wid * RPW
        pltpu.sync_copy(idx_hbm.at[pl.ds(base, RPW)], idx_v)
        pltpu.async_copy(table_hbm.at[idx_v], rows_v, sem).wait()
        pltpu.sync_copy(rows_v, out_hbm.at[pl.ds(base, RPW)])

    return k(idx_flat, mem_flat)


def _ln(x, g, b):
    mu = jnp.mean(x, axis=-1, keepdims=True)
    v = jnp.mean((x - mu) * (x - mu), axis=-1, keepdims=True)
    return (x - mu) / jnp.sqrt(v + 1e-5) * g + b


FCT = 256
NFC = D // FCT


def _fc_kernel(q_ref, r_ref, w_ref, b_ref, o_ref):
    wq = w_ref[0:D, :].astype(_BF)
    acc = jnp.dot(q_ref[...].astype(_BF), wq, preferred_element_type=jnp.float32)
    for k in range(K):
        wr = w_ref[(1 + k) * D:(2 + k) * D, :].astype(_BF)
        acc = acc + jnp.dot(r_ref[k * BS:(k + 1) * BS, :].astype(_BF), wr,
                            preferred_element_type=jnp.float32)
    o_ref[...] = acc + b_ref[...]


def _fc(q2d, r2d, w, bias):
    return pl.pallas_call(
        _fc_kernel,
        grid=(NFC,),
        in_specs=[
            pl.BlockSpec((BS, D), lambda j: (0, 0)),
            pl.BlockSpec((K * BS, D), lambda j: (0, 0)),
            pl.BlockSpec(((K + 1) * D, FCT), lambda j: (0, j)),
            pl.BlockSpec((FCT,), lambda j: (j,)),
        ],
        out_specs=pl.BlockSpec((BS, FCT), lambda j: (0, j)),
        out_shape=jax.ShapeDtypeStruct((BS, D), jnp.float32),
        compiler_params=pltpu.CompilerParams(
            dimension_semantics=("arbitrary",)),
    )(q2d, r2d, w, bias)


def _attn_kernel(x_ref, g1, be1, wqkv, bqkv, wo, bo, o_ref, att_sc, osc):
    x = x_ref[...]
    h = _ln(x, g1[...], be1[...])
    qkv = jnp.dot(h.astype(_BF), wqkv[...].astype(_BF),
                  preferred_element_type=jnp.float32) + bqkv[...]
    qkv_b = qkv.astype(_BF)

    for i in range(NH):
        q = qkv_b[:, i * DH:(i + 1) * DH]
        kk = qkv_b[:, D + i * DH:D + (i + 1) * DH]
        full = lax.dot_general(q, kk, (((1,), (1,)), ((), ())),
                               preferred_element_type=jnp.float32)
        for b in range(B):
            att_sc[b * S:(b + 1) * S, i * S:(i + 1) * S] = (
                full[b * S:(b + 1) * S, b * S:(b + 1) * S])

    ri = lax.broadcasted_iota(jnp.int32, (BS, S), 0)
    cij = lax.broadcasted_iota(jnp.int32, (BS, S), 1)
    causal = (ri % S) >= cij
    for i in range(NH):
        att = att_sc[:, i * S:(i + 1) * S] / 8.0
        att = jnp.where(causal, att, jnp.float32(-1e9))
        att = att - jnp.max(att, axis=-1, keepdims=True)
        e = jnp.exp(att)
        att_sc[:, i * S:(i + 1) * S] = e / jnp.sum(e, axis=-1, keepdims=True)

    for i in range(NH):
        for b in range(B):
            p = att_sc[b * S:(b + 1) * S, i * S:(i + 1) * S].astype(_BF)
            v = qkv_b[b * S:(b + 1) * S, 2 * D + i * DH:2 * D + (i + 1) * DH]
            osc[b * S:(b + 1) * S, i * DH:(i + 1) * DH] = jnp.dot(
                p, v, preferred_element_type=jnp.float32)

    o_ref[...] = x + jnp.dot(osc[...].astype(_BF), wo[...].astype(_BF),
                             preferred_element_type=jnp.float32) + bo[...]


HT = 768
NHC = (4 * D) // HT


def _mlp_kernel(x_ref, g2, be2, w1, b1, w2, b2, o_ref, hsc):
    j = pl.program_id(0)

    @pl.when(j == 0)
    def _():
        hsc[...] = _ln(x_ref[...], g2[...], be2[...]).astype(_BF)
        o_ref[...] = x_ref[...] + b2[...]

    hh = jnp.dot(hsc[...], w1[...].astype(_BF),
                 preferred_element_type=jnp.float32) + b1[0]
    hh = jax.nn.gelu(hh)
    o_ref[...] += jnp.dot(hh.astype(_BF), w2[...].astype(_BF),
                          preferred_element_type=jnp.float32)


def _block(x, p):
    x = pl.pallas_call(
        _attn_kernel,
        out_shape=jax.ShapeDtypeStruct((BS, D), jnp.float32),
        scratch_shapes=[pltpu.VMEM((BS, NH * S), jnp.float32),
                        pltpu.VMEM((BS, D), jnp.float32)],
    )(x, p["ln1_g"], p["ln1_b"], p["Wqkv"], p["bqkv"], p["Wo"], p["bo"])

    x = pl.pallas_call(
        _mlp_kernel,
        grid=(NHC,),
        in_specs=[
            pl.BlockSpec((BS, D), lambda j: (0, 0)),
            pl.BlockSpec((D,), lambda j: (0,)),
            pl.BlockSpec((D,), lambda j: (0,)),
            pl.BlockSpec((D, HT), lambda j: (0, j)),
            pl.BlockSpec((1, 1, HT), lambda j: (j, 0, 0)),
            pl.BlockSpec((HT, D), lambda j: (j, 0)),
            pl.BlockSpec((D,), lambda j: (0,)),
        ],
        out_specs=pl.BlockSpec((BS, D), lambda j: (0, 0)),
        out_shape=jax.ShapeDtypeStruct((BS, D), jnp.float32),
        scratch_shapes=[pltpu.VMEM((BS, D), _BF)],
        compiler_params=pltpu.CompilerParams(
            dimension_semantics=("arbitrary",)),
    )(x, p["ln2_g"], p["ln2_b"], p["W1"], p["b1"].reshape(NHC, 1, HT),
      p["W2"], p["b2"])
    return x


def _lm_kernel(x_ref, g, bb, wt_ref, o_ref):
    h = _ln(x_ref[...], g[...], bb[...])
    ot = lax.dot_general(wt_ref[...].astype(_BF), h.astype(_BF),
                         (((1,), (1,)), ((), ())),
                         preferred_element_type=jnp.float32)
    o_ref[...] = ot.reshape(VT, B, S)


def _lm_head(x2d, g, bb, wlm_t):
    return pl.pallas_call(
        _lm_kernel,
        grid=(NV,),
        in_specs=[
            pl.BlockSpec((BS, D), lambda j: (0, 0)),
            pl.BlockSpec((D,), lambda j: (0,)),
            pl.BlockSpec((D,), lambda j: (0,)),
            pl.BlockSpec((VT, D), lambda j: (j, 0)),
        ],
        out_specs=pl.BlockSpec((VT, B, S), lambda j: (j, 0, 0)),
        out_shape=jax.ShapeDtypeStruct((V, B, S), jnp.float32),
        compiler_params=pltpu.CompilerParams(
            dimension_semantics=("arbitrary",)),
    )(x2d, g, bb, wlm_t)


def kernel(input_vectors, memory, params):
    p = params
    idx_pad = _knn_topk(input_vectors, memory)
    idx_flat = idx_pad[:, :K, :].transpose(1, 0, 2).reshape(ROWS)

    retrieved = _gather_rows(idx_flat, memory.reshape(B * M, D))
    q2d = input_vectors.reshape(BS, D)

    x = _fc(q2d, retrieved, p["W_fc"], p["b_fc"])
    for bp in p["blocks"]:
        x = _block(x, bp)

    logits_t = _lm_head(x, p["lnf_g"], p["lnf_b"],
                        jnp.transpose(p["W_lm"]))
    return jnp.transpose(logits_t, (1, 2, 0))
